# baseline (speedup 1.0000x reference)
import jax
import jax.numpy as jnp
from jax.experimental import pallas as pl


def _dot(a, b):
    return jax.lax.dot_general(a, b, (((a.ndim - 1,), (0,)), ((), ())),
                               preferred_element_type=jnp.float32)


def _dot_t(a, b):
    return jax.lax.dot_general(a, b, (((1,), (1,)), ((), ())),
                               preferred_element_type=jnp.float32)


def _topk_max(ts, t_rows, topk):
    n = ts.shape[0]
    sim = _dot_t(ts, ts)
    col = jax.lax.broadcasted_iota(jnp.int32, (n, n), 1)
    acc = jnp.full(t_rows.shape, -jnp.inf, dtype=jnp.float32)
    cur = sim
    for _ in range(topk):
        m = jnp.max(cur, axis=1, keepdims=True)
        idx = jnp.min(jnp.where(cur == m, col, n), axis=1, keepdims=True)
        hit = col == idx
        acc = jnp.maximum(acc, _dot(hit.astype(jnp.float32), t_rows))
        cur = jnp.where(hit, -jnp.inf, cur)
    return acc


def _topk_max_stacked(t, topk, nper):
    rows = t.shape[0]
    sim = _dot_t(t, t)
    col = jax.lax.broadcasted_iota(jnp.int32, (rows, rows), 1)
    row = jax.lax.broadcasted_iota(jnp.int32, (rows, rows), 0)
    cur = jnp.where(row // nper == col // nper, sim, -jnp.inf)
    acc = jnp.full(t.shape, -jnp.inf, dtype=jnp.float32)
    for _ in range(topk):
        m = jnp.max(cur, axis=1, keepdims=True)
        idx = jnp.min(jnp.where(cur == m, col, rows), axis=1, keepdims=True)
        hit = col == idx
        acc = jnp.maximum(acc, _dot(hit.astype(jnp.float32), t))
        cur = jnp.where(hit, -jnp.inf, cur)
    return acc


def _tcg(patches, Wp, Wc, Wo, topk, x2, nper, stacked=False):
    t = _dot(patches, Wp)
    if x2 is not None:
        t = t + x2
    rows, d = t.shape
    if stacked:
        acc = _topk_max_stacked(t, topk, nper)
    else:
        accs = []
        for s in range(0, rows, nper):
            ts = t[s:s + nper]
            accs.append(_topk_max(ts, ts, topk))
        acc = jnp.concatenate(accs, axis=0) if len(accs) > 1 else accs[0]
    rel = acc - t
    h = _dot(t, Wc[:d]) + _dot(rel, Wc[d:])
    t_out = jax.nn.gelu(h) + t
    return _dot(t_out, Wo), t_out


_S = 4


def _stem_body(p_ref, w_ref, o_ref):
    o_ref[...] = jax.nn.gelu(
        _dot(p_ref[...].reshape(_S * 3136, 12), w_ref[...])
    ).reshape(_S, 3136, 46)


def _stage12_body(p_ref, d1p, d1c, d1o, s1p, s1c, s1o,
                  d2p, d2c, d2o, s2p, s2c, s2o, o_ref):
    p = p_ref[...].reshape(_S * 196, 736)
    pix, x2 = _tcg(p, d1p[...], d1c[...], d1o[...], 2, None, 196, True)
    for i in range(5):
        pix, x2 = _tcg(pix, s1p[i], s1c[i], s1o[i], 2, x2, 196, True)
    pix, x2 = _tcg(pix, d2p[...], d2c[...], d2o[...], 2, None, 196, True)
    for i in range(2):
        pix, x2 = _tcg(pix, s2p[i], s2c[i], s2o[i], 2, x2, 196, True)
    o_ref[...] = pix.reshape(_S, 196, 192)


def _stage3_body(p_ref, s21p, s21c, s21o, d3p, d3c, d3o, s3p, s3c, s3o,
                 fc, bg, bb, w1, b1, w2, b2, o_ref):
    p = p_ref[...].reshape(_S * 49, 768)
    pix, x2 = _tcg(p, s21p[...], s21c[...], s21o[...], 2, None, 49, True)
    pix, x2 = _tcg(pix, d3p[...], d3c[...], d3o[...], 9, None, 49, True)
    for i in range(2):
        pix, x2 = _tcg(pix, s3p[i], s3c[i], s3o[i], 2, x2, 49, True)
    f = _dot(pix, fc[...])
    f = f * bg[...] + bb[...]
    f = f * jax.nn.sigmoid(f)
    fm = jnp.mean(f.reshape(_S, 49, 384), axis=1)
    h2 = jax.nn.gelu(_dot(fm, w1[...]) + b1[...])
    o_ref[...] = (_dot(h2, w2[...]) + b2[...]).reshape(_S, 1, 250)


def _full(shape):
    nd = len(shape)
    return pl.BlockSpec(shape, lambda b, _nd=nd: (0,) * _nd)


def kernel(x, stem_W, ds1_Wp, ds1_Wc, ds1_Wo, s1_Wp, s1_Wc, s1_Wo,
           ds2_Wp, ds2_Wc, ds2_Wo, s20_Wp, s20_Wc, s20_Wo,
           s21_Wp, s21_Wc, s21_Wo, ds3_Wp, ds3_Wc, ds3_Wo,
           s3_Wp, s3_Wc, s3_Wo, fc_W, bn_g, bn_b,
           head_W1, head_b1, head_W2, head_b2):
    B = x.shape[0]
    f32 = jnp.float32

    sp = x.reshape(B, 3, 56, 2, 56, 2).transpose(0, 2, 4, 1, 3, 5)
    sp = sp.reshape(B, 3136, 12)
    h = pl.pallas_call(
        _stem_body,
        grid=(B // _S,),
        in_specs=[pl.BlockSpec((_S, 3136, 12), lambda b: (b, 0, 0)),
                  _full((12, 46))],
        out_specs=pl.BlockSpec((_S, 3136, 46), lambda b: (b, 0, 0)),
        out_shape=jax.ShapeDtypeStruct((B, 3136, 46), f32),
    )(sp, stem_W)

    p1 = h.reshape(B, 14, 4, 14, 4, 46).transpose(0, 1, 3, 5, 2, 4)
    p1 = p1.reshape(B, 196, 736)

    mid = pl.pallas_call(
        _stage12_body,
        grid=(B // _S,),
        in_specs=[pl.BlockSpec((_S, 196, 736), lambda b: (b, 0, 0)),
                  _full(ds1_Wp.shape), _full(ds1_Wc.shape), _full(ds1_Wo.shape),
                  _full(s1_Wp.shape), _full(s1_Wc.shape), _full(s1_Wo.shape),
                  _full(ds2_Wp.shape), _full(ds2_Wc.shape), _full(ds2_Wo.shape),
                  _full(s20_Wp.shape), _full(s20_Wc.shape), _full(s20_Wo.shape)],
        out_specs=pl.BlockSpec((_S, 196, 192), lambda b: (b, 0, 0)),
        out_shape=jax.ShapeDtypeStruct((B, 196, 192), f32),
    )(p1, ds1_Wp, ds1_Wc, ds1_Wo, s1_Wp, s1_Wc, s1_Wo,
      ds2_Wp, ds2_Wc, ds2_Wo, s20_Wp, s20_Wc, s20_Wo)

    p3 = mid.reshape(B, 7, 2, 7, 2, 192).transpose(0, 1, 3, 5, 2, 4)
    p3 = p3.reshape(B, 49, 768)

    out = pl.pallas_call(
        _stage3_body,
        grid=(B // _S,),
        in_specs=[pl.BlockSpec((_S, 49, 768), lambda b: (b, 0, 0)),
                  _full(s21_Wp.shape), _full(s21_Wc.shape), _full(s21_Wo.shape),
                  _full(ds3_Wp.shape), _full(ds3_Wc.shape), _full(ds3_Wo.shape),
                  _full(s3_Wp.shape), _full(s3_Wc.shape), _full(s3_Wo.shape),
                  _full(fc_W.shape), _full((1, 384)), _full((1, 384)),
                  _full(head_W1.shape), _full((1, 1536)),
                  _full(head_W2.shape), _full((1, 250))],
        out_specs=pl.BlockSpec((_S, 1, 250), lambda b: (b, 0, 0)),
        out_shape=jax.ShapeDtypeStruct((B, 1, 250), f32),
    )(p3, s21_Wp, s21_Wc, s21_Wo, ds3_Wp, ds3_Wc, ds3_Wo,
      s3_Wp, s3_Wc, s3_Wo, fc_W, bn_g.reshape(1, 384), bn_b.reshape(1, 384),
      head_W1, head_b1.reshape(1, 1536), head_W2, head_b2.reshape(1, 250))

    return out.reshape(B, 250)


# speedup vs baseline: 1.0550x; 1.0550x over previous
import jax
import jax.numpy as jnp
from jax.experimental import pallas as pl


def _dot(a, b):
    return jax.lax.dot_general(a, b, (((a.ndim - 1,), (0,)), ((), ())),
                               preferred_element_type=jnp.float32)


def _dot_t(a, b):
    return jax.lax.dot_general(a, b, (((1,), (1,)), ((), ())),
                               preferred_element_type=jnp.float32)


def _topk_max(ts, t_rows, topk):
    n = ts.shape[0]
    sim = _dot_t(ts, ts)
    col = jax.lax.broadcasted_iota(jnp.int32, (n, n), 1)
    acc = jnp.full(t_rows.shape, -jnp.inf, dtype=jnp.float32)
    cur = sim
    for _ in range(topk):
        m = jnp.max(cur, axis=1, keepdims=True)
        idx = jnp.min(jnp.where(cur == m, col, n), axis=1, keepdims=True)
        hit = col == idx
        acc = jnp.maximum(acc, _dot(hit.astype(jnp.float32), t_rows))
        cur = jnp.where(hit, -jnp.inf, cur)
    return acc


def _topk_max_stacked(t, topk, nper):
    rows = t.shape[0]
    sim = _dot_t(t, t)
    col = jax.lax.broadcasted_iota(jnp.int32, (rows, rows), 1)
    row = jax.lax.broadcasted_iota(jnp.int32, (rows, rows), 0)
    cur = jnp.where(row // nper == col // nper, sim, -jnp.inf)
    acc = jnp.full(t.shape, -jnp.inf, dtype=jnp.float32)
    for _ in range(topk):
        m = jnp.max(cur, axis=1, keepdims=True)
        idx = jnp.min(jnp.where(cur == m, col, rows), axis=1, keepdims=True)
        hit = col == idx
        acc = jnp.maximum(acc, _dot(hit.astype(jnp.float32), t))
        cur = jnp.where(hit, -jnp.inf, cur)
    return acc


def _tcg(patches, Wp, Wc, Wo, topk, x2, nper, stacked=False):
    t = _dot(patches, Wp)
    if x2 is not None:
        t = t + x2
    rows, d = t.shape
    if stacked:
        acc = _topk_max_stacked(t, topk, nper)
    else:
        accs = []
        for s in range(0, rows, nper):
            ts = t[s:s + nper]
            accs.append(_topk_max(ts, ts, topk))
        acc = jnp.concatenate(accs, axis=0) if len(accs) > 1 else accs[0]
    rel = acc - t
    h = _dot(t, Wc[:d]) + _dot(rel, Wc[d:])
    t_out = jax.nn.gelu(h) + t
    return _dot(t_out, Wo), t_out


_S1 = 4
_S2 = 8
_S3 = 4


def _stem_body(p_ref, w_ref, o_ref):
    s = p_ref.shape[0]
    o_ref[...] = jax.nn.gelu(
        _dot(p_ref[...].reshape(s * 3136, 12), w_ref[...])
    ).reshape(s, 3136, 46)


def _stage12_body(p_ref, d1p, d1c, d1o, s1p, s1c, s1o,
                  d2p, d2c, d2o, s2p, s2c, s2o, o_ref):
    s = p_ref.shape[0]
    p = p_ref[...].reshape(s * 196, 736)
    pix, x2 = _tcg(p, d1p[...], d1c[...], d1o[...], 2, None, 196)
    for i in range(5):
        pix, x2 = _tcg(pix, s1p[i], s1c[i], s1o[i], 2, x2, 196)
    pix, x2 = _tcg(pix, d2p[...], d2c[...], d2o[...], 2, None, 196)
    for i in range(2):
        pix, x2 = _tcg(pix, s2p[i], s2c[i], s2o[i], 2, x2, 196)
    o_ref[...] = pix.reshape(s, 196, 192)


def _stage3_body(p_ref, s21p, s21c, s21o, d3p, d3c, d3o, s3p, s3c, s3o,
                 fc, bg, bb, w1, b1, w2, b2, o_ref):
    s = p_ref.shape[0]
    p = p_ref[...].reshape(s * 49, 768)
    pix, x2 = _tcg(p, s21p[...], s21c[...], s21o[...], 2, None, 49, True)
    pix, x2 = _tcg(pix, d3p[...], d3c[...], d3o[...], 9, None, 49, True)
    for i in range(2):
        pix, x2 = _tcg(pix, s3p[i], s3c[i], s3o[i], 2, x2, 49, True)
    f = _dot(pix, fc[...])
    f = f * bg[...] + bb[...]
    f = f * jax.nn.sigmoid(f)
    fm = jnp.mean(f.reshape(s, 49, 384), axis=1)
    h2 = jax.nn.gelu(_dot(fm, w1[...]) + b1[...])
    o_ref[...] = (_dot(h2, w2[...]) + b2[...]).reshape(s, 1, 250)


def _full(shape):
    nd = len(shape)
    return pl.BlockSpec(shape, lambda b, _nd=nd: (0,) * _nd)


def kernel(x, stem_W, ds1_Wp, ds1_Wc, ds1_Wo, s1_Wp, s1_Wc, s1_Wo,
           ds2_Wp, ds2_Wc, ds2_Wo, s20_Wp, s20_Wc, s20_Wo,
           s21_Wp, s21_Wc, s21_Wo, ds3_Wp, ds3_Wc, ds3_Wo,
           s3_Wp, s3_Wc, s3_Wo, fc_W, bn_g, bn_b,
           head_W1, head_b1, head_W2, head_b2):
    B = x.shape[0]
    f32 = jnp.float32

    sp = x.reshape(B, 3, 56, 2, 56, 2).transpose(0, 2, 4, 1, 3, 5)
    sp = sp.reshape(B, 3136, 12)
    h = pl.pallas_call(
        _stem_body,
        grid=(B // _S1,),
        in_specs=[pl.BlockSpec((_S1, 3136, 12), lambda b: (b, 0, 0)),
                  _full((12, 46))],
        out_specs=pl.BlockSpec((_S1, 3136, 46), lambda b: (b, 0, 0)),
        out_shape=jax.ShapeDtypeStruct((B, 3136, 46), f32),
    )(sp, stem_W)

    p1 = h.reshape(B, 14, 4, 14, 4, 46).transpose(0, 1, 3, 5, 2, 4)
    p1 = p1.reshape(B, 196, 736)

    mid = pl.pallas_call(
        _stage12_body,
        grid=(B // _S2,),
        in_specs=[pl.BlockSpec((_S2, 196, 736), lambda b: (b, 0, 0)),
                  _full(ds1_Wp.shape), _full(ds1_Wc.shape), _full(ds1_Wo.shape),
                  _full(s1_Wp.shape), _full(s1_Wc.shape), _full(s1_Wo.shape),
                  _full(ds2_Wp.shape), _full(ds2_Wc.shape), _full(ds2_Wo.shape),
                  _full(s20_Wp.shape), _full(s20_Wc.shape), _full(s20_Wo.shape)],
        out_specs=pl.BlockSpec((_S2, 196, 192), lambda b: (b, 0, 0)),
        out_shape=jax.ShapeDtypeStruct((B, 196, 192), f32),
    )(p1, ds1_Wp, ds1_Wc, ds1_Wo, s1_Wp, s1_Wc, s1_Wo,
      ds2_Wp, ds2_Wc, ds2_Wo, s20_Wp, s20_Wc, s20_Wo)

    p3 = mid.reshape(B, 7, 2, 7, 2, 192).transpose(0, 1, 3, 5, 2, 4)
    p3 = p3.reshape(B, 49, 768)

    out = pl.pallas_call(
        _stage3_body,
        grid=(B // _S3,),
        in_specs=[pl.BlockSpec((_S3, 49, 768), lambda b: (b, 0, 0)),
                  _full(s21_Wp.shape), _full(s21_Wc.shape), _full(s21_Wo.shape),
                  _full(ds3_Wp.shape), _full(ds3_Wc.shape), _full(ds3_Wo.shape),
                  _full(s3_Wp.shape), _full(s3_Wc.shape), _full(s3_Wo.shape),
                  _full(fc_W.shape), _full((1, 384)), _full((1, 384)),
                  _full(head_W1.shape), _full((1, 1536)),
                  _full(head_W2.shape), _full((1, 250))],
        out_specs=pl.BlockSpec((_S3, 1, 250), lambda b: (b, 0, 0)),
        out_shape=jax.ShapeDtypeStruct((B, 1, 250), f32),
    )(p3, s21_Wp, s21_Wc, s21_Wo, ds3_Wp, ds3_Wc, ds3_Wo,
      s3_Wp, s3_Wc, s3_Wo, fc_W, bn_g.reshape(1, 384), bn_b.reshape(1, 384),
      head_W1, head_b1.reshape(1, 1536), head_W2, head_b2.reshape(1, 250))

    return out.reshape(B, 250)


# speedup vs baseline: 1.2129x; 1.1497x over previous
import jax
import jax.numpy as jnp
from jax.experimental import pallas as pl


def _dot(a, b):
    return jax.lax.dot_general(a, b, (((a.ndim - 1,), (0,)), ((), ())),
                               preferred_element_type=jnp.float32)


def _dot_t(a, b):
    return jax.lax.dot_general(a, b, (((1,), (1,)), ((), ())),
                               preferred_element_type=jnp.float32)


def _topk_max(ts, t_rows, topk):
    n = ts.shape[0]
    sim = _dot_t(ts, ts)
    col = jax.lax.broadcasted_iota(jnp.int32, (n, n), 1)
    acc = jnp.full(t_rows.shape, -jnp.inf, dtype=jnp.float32)
    cur = sim
    for _ in range(topk):
        m = jnp.max(cur, axis=1, keepdims=True)
        idx = jnp.min(jnp.where(cur == m, col, n), axis=1, keepdims=True)
        hit = col == idx
        acc = jnp.maximum(acc, _dot(hit.astype(jnp.float32), t_rows))
        cur = jnp.where(hit, -jnp.inf, cur)
    return acc


def _topk_max_stacked(t, topk, nper):
    rows = t.shape[0]
    sim = _dot_t(t, t)
    col = jax.lax.broadcasted_iota(jnp.int32, (rows, rows), 1)
    row = jax.lax.broadcasted_iota(jnp.int32, (rows, rows), 0)
    cur = jnp.where(row // nper == col // nper, sim, -jnp.inf)
    acc = jnp.full(t.shape, -jnp.inf, dtype=jnp.float32)
    for _ in range(topk):
        m = jnp.max(cur, axis=1, keepdims=True)
        idx = jnp.min(jnp.where(cur == m, col, rows), axis=1, keepdims=True)
        hit = col == idx
        acc = jnp.maximum(acc, _dot(hit.astype(jnp.float32), t))
        cur = jnp.where(hit, -jnp.inf, cur)
    return acc


def _tcg(patches, Wp, Wc, Wo, topk, x2, nper, stacked=False):
    t = _dot(patches, Wp)
    if x2 is not None:
        t = t + x2
    rows, d = t.shape
    if stacked:
        acc = _topk_max_stacked(t, topk, nper)
    else:
        accs = []
        for s in range(0, rows, nper):
            ts = t[s:s + nper]
            accs.append(_topk_max(ts, ts, topk))
        acc = jnp.concatenate(accs, axis=0) if len(accs) > 1 else accs[0]
    rel = acc - t
    h = _dot(t, Wc[:d]) + _dot(rel, Wc[d:])
    t_out = jax.nn.gelu(h) + t
    return _dot(t_out, Wo), t_out


_S1 = 4
_S2 = 4
_S3 = 4


def _stem_body(p_ref, w_ref, o_ref):
    s = p_ref.shape[0]
    o_ref[...] = jax.nn.gelu(
        _dot(p_ref[...].reshape(s * 3136, 12), w_ref[...])
    ).reshape(s, 196, 16, 46)


def _stage12_body(p_ref, d1p, d1c, d1o, s1p, s1c, s1o,
                  d2p, d2c, d2o, s2p, s2c, s2o, o_ref):
    s = p_ref.shape[0]
    p = p_ref[...].reshape(s * 196, 736)
    pix, x2 = _tcg(p, d1p[...], d1c[...], d1o[...], 2, None, 196)
    for i in range(5):
        pix, x2 = _tcg(pix, s1p[i], s1c[i], s1o[i], 2, x2, 196)
    pix, x2 = _tcg(pix, d2p[...], d2c[...], d2o[...], 2, None, 196)
    for i in range(2):
        pix, x2 = _tcg(pix, s2p[i], s2c[i], s2o[i], 2, x2, 196)
    o_ref[...] = pix.reshape(s, 196, 192)


def _stage3_body(p_ref, s21p, s21c, s21o, d3p, d3c, d3o, s3p, s3c, s3o,
                 fc, bg, bb, w1, b1, w2, b2, o_ref):
    s = p_ref.shape[0]
    p = p_ref[...].reshape(s * 49, 768)
    pix, x2 = _tcg(p, s21p[...], s21c[...], s21o[...], 2, None, 49, True)
    pix, x2 = _tcg(pix, d3p[...], d3c[...], d3o[...], 9, None, 49, True)
    for i in range(2):
        pix, x2 = _tcg(pix, s3p[i], s3c[i], s3o[i], 2, x2, 49, True)
    f = _dot(pix, fc[...])
    f = f * bg[...] + bb[...]
    f = f * jax.nn.sigmoid(f)
    fm = jnp.mean(f.reshape(s, 49, 384), axis=1)
    h2 = jax.nn.gelu(_dot(fm, w1[...]) + b1[...])
    o_ref[...] = (_dot(h2, w2[...]) + b2[...]).reshape(s, 1, 250)


def _full(shape):
    nd = len(shape)
    return pl.BlockSpec(shape, lambda b, _nd=nd: (0,) * _nd)


def kernel(x, stem_W, ds1_Wp, ds1_Wc, ds1_Wo, s1_Wp, s1_Wc, s1_Wo,
           ds2_Wp, ds2_Wc, ds2_Wo, s20_Wp, s20_Wc, s20_Wo,
           s21_Wp, s21_Wc, s21_Wo, ds3_Wp, ds3_Wc, ds3_Wo,
           s3_Wp, s3_Wc, s3_Wo, fc_W, bn_g, bn_b,
           head_W1, head_b1, head_W2, head_b2):
    B = x.shape[0]
    f32 = jnp.float32

    sp = x.reshape(B, 3, 14, 4, 2, 14, 4, 2).transpose(0, 2, 5, 3, 6, 1, 4, 7)
    sp = sp.reshape(B, 196, 16, 12)
    wp1 = ds1_Wp.reshape(46, 16, 92).transpose(1, 0, 2).reshape(736, 92)
    h = pl.pallas_call(
        _stem_body,
        grid=(B // _S1,),
        in_specs=[pl.BlockSpec((_S1, 196, 16, 12), lambda b: (b, 0, 0, 0)),
                  _full((12, 46))],
        out_specs=pl.BlockSpec((_S1, 196, 16, 46), lambda b: (b, 0, 0, 0)),
        out_shape=jax.ShapeDtypeStruct((B, 196, 16, 46), f32),
    )(sp, stem_W)

    p1 = h.reshape(B, 196, 736)

    mid = pl.pallas_call(
        _stage12_body,
        grid=(B // _S2,),
        in_specs=[pl.BlockSpec((_S2, 196, 736), lambda b: (b, 0, 0)),
                  _full(ds1_Wp.shape), _full(ds1_Wc.shape), _full(ds1_Wo.shape),
                  _full(s1_Wp.shape), _full(s1_Wc.shape), _full(s1_Wo.shape),
                  _full(ds2_Wp.shape), _full(ds2_Wc.shape), _full(ds2_Wo.shape),
                  _full(s20_Wp.shape), _full(s20_Wc.shape), _full(s20_Wo.shape)],
        out_specs=pl.BlockSpec((_S2, 196, 192), lambda b: (b, 0, 0)),
        out_shape=jax.ShapeDtypeStruct((B, 196, 192), f32),
    )(p1, wp1, ds1_Wc, ds1_Wo, s1_Wp, s1_Wc, s1_Wo,
      ds2_Wp, ds2_Wc, ds2_Wo, s20_Wp, s20_Wc, s20_Wo)

    p3 = mid.reshape(B, 7, 2, 7, 2, 192).transpose(0, 1, 3, 5, 2, 4)
    p3 = p3.reshape(B, 49, 768)

    out = pl.pallas_call(
        _stage3_body,
        grid=(B // _S3,),
        in_specs=[pl.BlockSpec((_S3, 49, 768), lambda b: (b, 0, 0)),
                  _full(s21_Wp.shape), _full(s21_Wc.shape), _full(s21_Wo.shape),
                  _full(ds3_Wp.shape), _full(ds3_Wc.shape), _full(ds3_Wo.shape),
                  _full(s3_Wp.shape), _full(s3_Wc.shape), _full(s3_Wo.shape),
                  _full(fc_W.shape), _full((1, 384)), _full((1, 384)),
                  _full(head_W1.shape), _full((1, 1536)),
                  _full(head_W2.shape), _full((1, 250))],
        out_specs=pl.BlockSpec((_S3, 1, 250), lambda b: (b, 0, 0)),
        out_shape=jax.ShapeDtypeStruct((B, 1, 250), f32),
    )(p3, s21_Wp, s21_Wc, s21_Wo, ds3_Wp, ds3_Wc, ds3_Wo,
      s3_Wp, s3_Wc, s3_Wo, fc_W, bn_g.reshape(1, 384), bn_b.reshape(1, 384),
      head_W1, head_b1.reshape(1, 1536), head_W2, head_b2.reshape(1, 250))

    return out.reshape(B, 250)


# speedup vs baseline: 1.2142x; 1.0010x over previous
import jax
import jax.numpy as jnp
from jax.experimental import pallas as pl
from jax.experimental.pallas import tpu as pltpu

_PAR = pltpu.CompilerParams(dimension_semantics=("parallel",))


def _dot(a, b):
    return jax.lax.dot_general(a, b, (((a.ndim - 1,), (0,)), ((), ())),
                               preferred_element_type=jnp.float32)


def _dot_t(a, b):
    return jax.lax.dot_general(a, b, (((1,), (1,)), ((), ())),
                               preferred_element_type=jnp.float32)


def _topk_max(ts, t_rows, topk):
    n = ts.shape[0]
    sim = _dot_t(ts, ts)
    col = jax.lax.broadcasted_iota(jnp.int32, (n, n), 1)
    acc = jnp.full(t_rows.shape, -jnp.inf, dtype=jnp.float32)
    cur = sim
    for _ in range(topk):
        m = jnp.max(cur, axis=1, keepdims=True)
        idx = jnp.min(jnp.where(cur == m, col, n), axis=1, keepdims=True)
        hit = col == idx
        acc = jnp.maximum(acc, _dot(hit.astype(jnp.float32), t_rows))
        cur = jnp.where(hit, -jnp.inf, cur)
    return acc


def _topk_max_stacked(t, topk, nper):
    rows = t.shape[0]
    sim = _dot_t(t, t)
    col = jax.lax.broadcasted_iota(jnp.int32, (rows, rows), 1)
    row = jax.lax.broadcasted_iota(jnp.int32, (rows, rows), 0)
    cur = jnp.where(row // nper == col // nper, sim, -jnp.inf)
    acc = jnp.full(t.shape, -jnp.inf, dtype=jnp.float32)
    for _ in range(topk):
        m = jnp.max(cur, axis=1, keepdims=True)
        idx = jnp.min(jnp.where(cur == m, col, rows), axis=1, keepdims=True)
        hit = col == idx
        acc = jnp.maximum(acc, _dot(hit.astype(jnp.float32), t))
        cur = jnp.where(hit, -jnp.inf, cur)
    return acc


def _tcg(patches, Wp, Wc, Wo, topk, x2, nper, stacked=False):
    t = _dot(patches, Wp)
    if x2 is not None:
        t = t + x2
    rows, d = t.shape
    if stacked:
        acc = _topk_max_stacked(t, topk, nper)
    else:
        accs = []
        for s in range(0, rows, nper):
            ts = t[s:s + nper]
            accs.append(_topk_max(ts, ts, topk))
        acc = jnp.concatenate(accs, axis=0) if len(accs) > 1 else accs[0]
    rel = acc - t
    h = _dot(t, Wc[:d]) + _dot(rel, Wc[d:])
    t_out = jax.nn.gelu(h) + t
    return _dot(t_out, Wo), t_out


_S1 = 4
_S2 = 4
_S3 = 4


def _stem_body(p_ref, w_ref, o_ref):
    s = p_ref.shape[0]
    o_ref[...] = jax.nn.gelu(
        _dot(p_ref[...].reshape(s * 3136, 12), w_ref[...])
    ).reshape(s, 196, 16, 46)


def _stage12_body(p_ref, d1p, d1c, d1o, s1p, s1c, s1o,
                  d2p, d2c, d2o, s2p, s2c, s2o, o_ref):
    s = p_ref.shape[0]
    p = p_ref[...].reshape(s * 196, 736)
    pix, x2 = _tcg(p, d1p[...], d1c[...], d1o[...], 2, None, 196)
    for i in range(5):
        pix, x2 = _tcg(pix, s1p[i], s1c[i], s1o[i], 2, x2, 196)
    pix, x2 = _tcg(pix, d2p[...], d2c[...], d2o[...], 2, None, 196)
    for i in range(2):
        pix, x2 = _tcg(pix, s2p[i], s2c[i], s2o[i], 2, x2, 196)
    o_ref[...] = pix.reshape(s, 196, 192)


def _stage3_body(p_ref, s21p, s21c, s21o, d3p, d3c, d3o, s3p, s3c, s3o,
                 fc, bg, bb, w1, b1, w2, b2, o_ref):
    s = p_ref.shape[0]
    p = p_ref[...].reshape(s * 49, 768)
    pix, x2 = _tcg(p, s21p[...], s21c[...], s21o[...], 2, None, 49, True)
    pix, x2 = _tcg(pix, d3p[...], d3c[...], d3o[...], 9, None, 49, True)
    for i in range(2):
        pix, x2 = _tcg(pix, s3p[i], s3c[i], s3o[i], 2, x2, 49, True)
    f = _dot(pix, fc[...])
    f = f * bg[...] + bb[...]
    f = f * jax.nn.sigmoid(f)
    fm = jnp.mean(f.reshape(s, 49, 384), axis=1)
    h2 = jax.nn.gelu(_dot(fm, w1[...]) + b1[...])
    o_ref[...] = (_dot(h2, w2[...]) + b2[...]).reshape(s, 1, 250)


def _full(shape):
    nd = len(shape)
    return pl.BlockSpec(shape, lambda b, _nd=nd: (0,) * _nd)


def kernel(x, stem_W, ds1_Wp, ds1_Wc, ds1_Wo, s1_Wp, s1_Wc, s1_Wo,
           ds2_Wp, ds2_Wc, ds2_Wo, s20_Wp, s20_Wc, s20_Wo,
           s21_Wp, s21_Wc, s21_Wo, ds3_Wp, ds3_Wc, ds3_Wo,
           s3_Wp, s3_Wc, s3_Wo, fc_W, bn_g, bn_b,
           head_W1, head_b1, head_W2, head_b2):
    B = x.shape[0]
    f32 = jnp.float32

    sp = x.reshape(B, 3, 14, 4, 2, 14, 4, 2).transpose(0, 2, 5, 3, 6, 1, 4, 7)
    sp = sp.reshape(B, 196, 16, 12)
    wp1 = ds1_Wp.reshape(46, 16, 92).transpose(1, 0, 2).reshape(736, 92)
    h = pl.pallas_call(
        _stem_body,
        compiler_params=_PAR,
        grid=(B // _S1,),
        in_specs=[pl.BlockSpec((_S1, 196, 16, 12), lambda b: (b, 0, 0, 0)),
                  _full((12, 46))],
        out_specs=pl.BlockSpec((_S1, 196, 16, 46), lambda b: (b, 0, 0, 0)),
        out_shape=jax.ShapeDtypeStruct((B, 196, 16, 46), f32),
    )(sp, stem_W)

    p1 = h.reshape(B, 196, 736)

    mid = pl.pallas_call(
        _stage12_body,
        compiler_params=_PAR,
        grid=(B // _S2,),
        in_specs=[pl.BlockSpec((_S2, 196, 736), lambda b: (b, 0, 0)),
                  _full(ds1_Wp.shape), _full(ds1_Wc.shape), _full(ds1_Wo.shape),
                  _full(s1_Wp.shape), _full(s1_Wc.shape), _full(s1_Wo.shape),
                  _full(ds2_Wp.shape), _full(ds2_Wc.shape), _full(ds2_Wo.shape),
                  _full(s20_Wp.shape), _full(s20_Wc.shape), _full(s20_Wo.shape)],
        out_specs=pl.BlockSpec((_S2, 196, 192), lambda b: (b, 0, 0)),
        out_shape=jax.ShapeDtypeStruct((B, 196, 192), f32),
    )(p1, wp1, ds1_Wc, ds1_Wo, s1_Wp, s1_Wc, s1_Wo,
      ds2_Wp, ds2_Wc, ds2_Wo, s20_Wp, s20_Wc, s20_Wo)

    p3 = mid.reshape(B, 7, 2, 7, 2, 192).transpose(0, 1, 3, 5, 2, 4)
    p3 = p3.reshape(B, 49, 768)

    out = pl.pallas_call(
        _stage3_body,
        compiler_params=_PAR,
        grid=(B // _S3,),
        in_specs=[pl.BlockSpec((_S3, 49, 768), lambda b: (b, 0, 0)),
                  _full(s21_Wp.shape), _full(s21_Wc.shape), _full(s21_Wo.shape),
                  _full(ds3_Wp.shape), _full(ds3_Wc.shape), _full(ds3_Wo.shape),
                  _full(s3_Wp.shape), _full(s3_Wc.shape), _full(s3_Wo.shape),
                  _full(fc_W.shape), _full((1, 384)), _full((1, 384)),
                  _full(head_W1.shape), _full((1, 1536)),
                  _full(head_W2.shape), _full((1, 250))],
        out_specs=pl.BlockSpec((_S3, 1, 250), lambda b: (b, 0, 0)),
        out_shape=jax.ShapeDtypeStruct((B, 1, 250), f32),
    )(p3, s21_Wp, s21_Wc, s21_Wo, ds3_Wp, ds3_Wc, ds3_Wo,
      s3_Wp, s3_Wc, s3_Wo, fc_W, bn_g.reshape(1, 384), bn_b.reshape(1, 384),
      head_W1, head_b1.reshape(1, 1536), head_W2, head_b2.reshape(1, 250))

    return out.reshape(B, 250)


# speedup vs baseline: 1.8040x; 1.4858x over previous
import jax
import jax.numpy as jnp
import numpy as np
from jax.experimental import pallas as pl
from jax.experimental.pallas import tpu as pltpu

_PAR = pltpu.CompilerParams(dimension_semantics=("parallel",))


def _dot(a, b):
    return jax.lax.dot_general(a, b, (((a.ndim - 1,), (0,)), ((), ())),
                               preferred_element_type=jnp.float32)


def _dot_t(a, b):
    return jax.lax.dot_general(a, b, (((1,), (1,)), ((), ())),
                               preferred_element_type=jnp.float32)


def _topk_max(ts, t_rows, topk):
    n = ts.shape[0]
    sim = _dot_t(ts, ts)
    col = jax.lax.broadcasted_iota(jnp.int32, (n, n), 1)
    acc = jnp.full(t_rows.shape, -jnp.inf, dtype=jnp.float32)
    cur = sim
    for _ in range(topk):
        m = jnp.max(cur, axis=1, keepdims=True)
        idx = jnp.min(jnp.where(cur == m, col, n), axis=1, keepdims=True)
        hit = col == idx
        acc = jnp.maximum(acc, _dot(hit.astype(jnp.float32), t_rows))
        cur = jnp.where(hit, -jnp.inf, cur)
    return acc


def _topk_max_stacked(t, topk, nper):
    rows = t.shape[0]
    sim = _dot_t(t, t)
    col = jax.lax.broadcasted_iota(jnp.int32, (rows, rows), 1)
    row = jax.lax.broadcasted_iota(jnp.int32, (rows, rows), 0)
    cur = jnp.where(row // nper == col // nper, sim, -jnp.inf)
    acc = jnp.full(t.shape, -jnp.inf, dtype=jnp.float32)
    for _ in range(topk):
        m = jnp.max(cur, axis=1, keepdims=True)
        idx = jnp.min(jnp.where(cur == m, col, rows), axis=1, keepdims=True)
        hit = col == idx
        acc = jnp.maximum(acc, _dot(hit.astype(jnp.float32), t))
        cur = jnp.where(hit, -jnp.inf, cur)
    return acc


def _tcg(patches, Wp, Wc, Wo, topk, x2, nper, stacked=False):
    t = _dot(patches, Wp)
    if x2 is not None:
        t = t + x2
    rows, d = t.shape
    if stacked:
        acc = _topk_max_stacked(t, topk, nper)
    else:
        accs = []
        for s in range(0, rows, nper):
            ts = t[s:s + nper]
            accs.append(_topk_max(ts, ts, topk))
        acc = jnp.concatenate(accs, axis=0) if len(accs) > 1 else accs[0]
    rel = acc - t
    h = _dot(t, Wc[:d]) + _dot(rel, Wc[d:])
    t_out = jax.nn.gelu(h) + t
    return _dot(t_out, Wo), t_out


_S2 = 4
_S3 = 4

_P196 = np.zeros((196, 196), dtype=np.float32)
for _a in range(7):
    for _b in range(7):
        for _pi in range(2):
            for _pj in range(2):
                _P196[_a * 28 + _b * 4 + _pi * 2 + _pj,
                      (2 * _a + _pi) * 14 + 2 * _b + _pj] = 1.0


def _stage12_body(p_ref, wd, d1p, d1c, d1o, s1p, s1c, s1o,
                  d2p, d2c, d2o, s2p, s2c, s2o, perm, o_ref):
    s = p_ref.shape[0]
    p = jax.nn.gelu(_dot(p_ref[...].reshape(s * 196, 192), wd[...]))
    pix, x2 = _tcg(p, d1p[...], d1c[...], d1o[...], 2, None, 196)
    for i in range(5):
        pix, x2 = _tcg(pix, s1p[i], s1c[i], s1o[i], 2, x2, 196)
    pix, x2 = _tcg(pix, d2p[...], d2c[...], d2o[...], 2, None, 196)
    for i in range(2):
        pix, x2 = _tcg(pix, s2p[i], s2c[i], s2o[i], 2, x2, 196)
    pm = perm[...]
    outs = [_dot(pm, pix[i * 196:(i + 1) * 196]) for i in range(s)]
    o_ref[...] = jnp.concatenate(outs, axis=0).reshape(s, 196, 192)


def _stage3_body(p_ref, s21p, s21c, s21o, d3p, d3c, d3o, s3p, s3c, s3o,
                 fc, bg, bb, w1, b1, w2, b2, o_ref):
    s = p_ref.shape[0]
    p = p_ref[...].reshape(s * 49, 768)
    pix, x2 = _tcg(p, s21p[...], s21c[...], s21o[...], 2, None, 49, True)
    pix, x2 = _tcg(pix, d3p[...], d3c[...], d3o[...], 9, None, 49, True)
    for i in range(2):
        pix, x2 = _tcg(pix, s3p[i], s3c[i], s3o[i], 2, x2, 49, True)
    f = _dot(pix, fc[...])
    f = f * bg[...] + bb[...]
    f = f * jax.nn.sigmoid(f)
    fm = jnp.mean(f.reshape(s, 49, 384), axis=1)
    h2 = jax.nn.gelu(_dot(fm, w1[...]) + b1[...])
    o_ref[...] = (_dot(h2, w2[...]) + b2[...]).reshape(s, 1, 250)


def _full(shape):
    nd = len(shape)
    return pl.BlockSpec(shape, lambda b, _nd=nd: (0,) * _nd)


def kernel(x, stem_W, ds1_Wp, ds1_Wc, ds1_Wo, s1_Wp, s1_Wc, s1_Wo,
           ds2_Wp, ds2_Wc, ds2_Wo, s20_Wp, s20_Wc, s20_Wo,
           s21_Wp, s21_Wc, s21_Wo, ds3_Wp, ds3_Wc, ds3_Wo,
           s3_Wp, s3_Wc, s3_Wo, fc_W, bn_g, bn_b,
           head_W1, head_b1, head_W2, head_b2):
    B = x.shape[0]
    f32 = jnp.float32

    sp = x.reshape(B, 3, 14, 4, 2, 14, 4, 2).transpose(0, 2, 5, 3, 6, 1, 4, 7)
    sp = sp.reshape(B, 196, 192)
    wd = jnp.kron(jnp.eye(16, dtype=f32), stem_W)
    wp1 = ds1_Wp.reshape(46, 16, 92).transpose(1, 0, 2).reshape(736, 92)

    mid = pl.pallas_call(
        _stage12_body,
        compiler_params=_PAR,
        grid=(B // _S2,),
        in_specs=[pl.BlockSpec((_S2, 196, 192), lambda b: (b, 0, 0)),
                  _full((192, 736)), _full(ds1_Wp.shape), _full(ds1_Wc.shape), _full(ds1_Wo.shape),
                  _full(s1_Wp.shape), _full(s1_Wc.shape), _full(s1_Wo.shape),
                  _full(ds2_Wp.shape), _full(ds2_Wc.shape), _full(ds2_Wo.shape),
                  _full(s20_Wp.shape), _full(s20_Wc.shape), _full(s20_Wo.shape),
                  _full((196, 196))],
        out_specs=pl.BlockSpec((_S2, 196, 192), lambda b: (b, 0, 0)),
        out_shape=jax.ShapeDtypeStruct((B, 196, 192), f32),
    )(sp, wd, wp1, ds1_Wc, ds1_Wo, s1_Wp, s1_Wc, s1_Wo,
      ds2_Wp, ds2_Wc, ds2_Wo, s20_Wp, s20_Wc, s20_Wo, jnp.asarray(_P196))

    p3 = mid.reshape(B, 49, 768)
    wp21 = s21_Wp.reshape(192, 4, 192).transpose(1, 0, 2).reshape(768, 192)

    out = pl.pallas_call(
        _stage3_body,
        compiler_params=_PAR,
        grid=(B // _S3,),
        in_specs=[pl.BlockSpec((_S3, 49, 768), lambda b: (b, 0, 0)),
                  _full(s21_Wp.shape), _full(s21_Wc.shape), _full(s21_Wo.shape),
                  _full(ds3_Wp.shape), _full(ds3_Wc.shape), _full(ds3_Wo.shape),
                  _full(s3_Wp.shape), _full(s3_Wc.shape), _full(s3_Wo.shape),
                  _full(fc_W.shape), _full((1, 384)), _full((1, 384)),
                  _full(head_W1.shape), _full((1, 1536)),
                  _full(head_W2.shape), _full((1, 250))],
        out_specs=pl.BlockSpec((_S3, 1, 250), lambda b: (b, 0, 0)),
        out_shape=jax.ShapeDtypeStruct((B, 1, 250), f32),
    )(p3, wp21, s21_Wc, s21_Wo, ds3_Wp, ds3_Wc, ds3_Wo,
      s3_Wp, s3_Wc, s3_Wo, fc_W, bn_g.reshape(1, 384), bn_b.reshape(1, 384),
      head_W1, head_b1.reshape(1, 1536), head_W2, head_b2.reshape(1, 250))

    return out.reshape(B, 250)


# speedup vs baseline: 2.0677x; 1.1462x over previous
import jax
import jax.numpy as jnp
import numpy as np
from jax.experimental import pallas as pl
from jax.experimental.pallas import tpu as pltpu

_PAR = pltpu.CompilerParams(dimension_semantics=("parallel",))


def _dot(a, b):
    return jax.lax.dot_general(a, b, (((a.ndim - 1,), (0,)), ((), ())),
                               preferred_element_type=jnp.float32)


def _dot_t(a, b):
    return jax.lax.dot_general(a, b, (((1,), (1,)), ((), ())),
                               preferred_element_type=jnp.float32)


def _topk_max(ts, t_rows, topk):
    n = ts.shape[0]
    sim = _dot_t(ts, ts)
    col = jax.lax.broadcasted_iota(jnp.int32, (n, n), 1).astype(jnp.float32)
    acc = jnp.full(t_rows.shape, -jnp.inf, dtype=jnp.float32)
    cur = sim
    for _ in range(topk):
        m = jnp.max(cur, axis=1, keepdims=True)
        w = jnp.where(cur == m, col, jnp.float32(n))
        idx = jnp.min(w, axis=1, keepdims=True)
        hit = w == idx
        acc = jnp.maximum(acc, _dot(jnp.where(hit, 1.0, 0.0), t_rows))
        cur = jnp.where(hit, -jnp.inf, cur)
    return acc


def _topk_max_stacked(t, topk, nper):
    rows = t.shape[0]
    sim = _dot_t(t, t)
    coli = jax.lax.broadcasted_iota(jnp.int32, (rows, rows), 1)
    col = coli.astype(jnp.float32)
    rowi = jax.lax.broadcasted_iota(jnp.int32, (rows, rows), 0)
    cur = jnp.where(rowi // nper == coli // nper, sim, -jnp.inf)
    acc = jnp.full(t.shape, -jnp.inf, dtype=jnp.float32)
    for _ in range(topk):
        m = jnp.max(cur, axis=1, keepdims=True)
        w = jnp.where(cur == m, col, jnp.float32(rows))
        idx = jnp.min(w, axis=1, keepdims=True)
        hit = w == idx
        acc = jnp.maximum(acc, _dot(jnp.where(hit, 1.0, 0.0), t))
        cur = jnp.where(hit, -jnp.inf, cur)
    return acc


def _tcg(patches, Wp, Wc, Wo, topk, x2, nper, stacked=False):
    t = _dot(patches, Wp)
    if x2 is not None:
        t = t + x2
    rows, d = t.shape
    if stacked:
        acc = _topk_max_stacked(t, topk, nper)
    else:
        accs = []
        for s in range(0, rows, nper):
            ts = t[s:s + nper]
            accs.append(_topk_max(ts, ts, topk))
        acc = jnp.concatenate(accs, axis=0) if len(accs) > 1 else accs[0]
    rel = acc - t
    h = _dot(t, Wc[:d]) + _dot(rel, Wc[d:])
    t_out = jax.nn.gelu(h) + t
    return _dot(t_out, Wo), t_out


_S2 = 4
_S3 = 4

_P196 = np.zeros((196, 196), dtype=np.float32)
for _a in range(7):
    for _b in range(7):
        for _pi in range(2):
            for _pj in range(2):
                _P196[_a * 28 + _b * 4 + _pi * 2 + _pj,
                      (2 * _a + _pi) * 14 + 2 * _b + _pj] = 1.0


def _stage12_body(p_ref, wd, d1p, d1c, d1o, s1p, s1c, s1o,
                  d2p, d2c, d2o, s2p, s2c, s2o, perm, o_ref):
    s = p_ref.shape[0]
    p = jax.nn.gelu(_dot(p_ref[...].reshape(s * 196, 192), wd[...]))
    pix, x2 = _tcg(p, d1p[...], d1c[...], d1o[...], 2, None, 196)
    for i in range(5):
        pix, x2 = _tcg(pix, s1p[i], s1c[i], s1o[i], 2, x2, 196)
    pix, x2 = _tcg(pix, d2p[...], d2c[...], d2o[...], 2, None, 196)
    for i in range(2):
        pix, x2 = _tcg(pix, s2p[i], s2c[i], s2o[i], 2, x2, 196)
    pm = perm[...]
    outs = [_dot(pm, pix[i * 196:(i + 1) * 196]) for i in range(s)]
    o_ref[...] = jnp.concatenate(outs, axis=0).reshape(s, 196, 192)


def _stage3_body(p_ref, s21p, s21c, s21o, d3p, d3c, d3o, s3p, s3c, s3o,
                 fc, bg, bb, w1, b1, w2, b2, o_ref):
    s = p_ref.shape[0]
    p = p_ref[...].reshape(s * 49, 768)
    pix, x2 = _tcg(p, s21p[...], s21c[...], s21o[...], 2, None, 49, True)
    pix, x2 = _tcg(pix, d3p[...], d3c[...], d3o[...], 9, None, 49, True)
    for i in range(2):
        pix, x2 = _tcg(pix, s3p[i], s3c[i], s3o[i], 2, x2, 49, True)
    f = _dot(pix, fc[...])
    f = f * bg[...] + bb[...]
    f = f * jax.nn.sigmoid(f)
    fm = jnp.mean(f.reshape(s, 49, 384), axis=1)
    h2 = jax.nn.gelu(_dot(fm, w1[...]) + b1[...])
    o_ref[...] = (_dot(h2, w2[...]) + b2[...]).reshape(s, 1, 250)


def _full(shape):
    nd = len(shape)
    return pl.BlockSpec(shape, lambda b, _nd=nd: (0,) * _nd)


def kernel(x, stem_W, ds1_Wp, ds1_Wc, ds1_Wo, s1_Wp, s1_Wc, s1_Wo,
           ds2_Wp, ds2_Wc, ds2_Wo, s20_Wp, s20_Wc, s20_Wo,
           s21_Wp, s21_Wc, s21_Wo, ds3_Wp, ds3_Wc, ds3_Wo,
           s3_Wp, s3_Wc, s3_Wo, fc_W, bn_g, bn_b,
           head_W1, head_b1, head_W2, head_b2):
    B = x.shape[0]
    f32 = jnp.float32

    sp = x.reshape(B, 3, 14, 4, 2, 14, 4, 2).transpose(0, 2, 5, 3, 6, 1, 4, 7)
    sp = sp.reshape(B, 196, 192)
    wd = jnp.kron(jnp.eye(16, dtype=f32), stem_W)
    wp1 = ds1_Wp.reshape(46, 16, 92).transpose(1, 0, 2).reshape(736, 92)

    mid = pl.pallas_call(
        _stage12_body,
        compiler_params=_PAR,
        grid=(B // _S2,),
        in_specs=[pl.BlockSpec((_S2, 196, 192), lambda b: (b, 0, 0)),
                  _full((192, 736)), _full(ds1_Wp.shape), _full(ds1_Wc.shape), _full(ds1_Wo.shape),
                  _full(s1_Wp.shape), _full(s1_Wc.shape), _full(s1_Wo.shape),
                  _full(ds2_Wp.shape), _full(ds2_Wc.shape), _full(ds2_Wo.shape),
                  _full(s20_Wp.shape), _full(s20_Wc.shape), _full(s20_Wo.shape),
                  _full((196, 196))],
        out_specs=pl.BlockSpec((_S2, 196, 192), lambda b: (b, 0, 0)),
        out_shape=jax.ShapeDtypeStruct((B, 196, 192), f32),
    )(sp, wd, wp1, ds1_Wc, ds1_Wo, s1_Wp, s1_Wc, s1_Wo,
      ds2_Wp, ds2_Wc, ds2_Wo, s20_Wp, s20_Wc, s20_Wo, jnp.asarray(_P196))

    p3 = mid.reshape(B, 49, 768)
    wp21 = s21_Wp.reshape(192, 4, 192).transpose(1, 0, 2).reshape(768, 192)

    out = pl.pallas_call(
        _stage3_body,
        compiler_params=_PAR,
        grid=(B // _S3,),
        in_specs=[pl.BlockSpec((_S3, 49, 768), lambda b: (b, 0, 0)),
                  _full(s21_Wp.shape), _full(s21_Wc.shape), _full(s21_Wo.shape),
                  _full(ds3_Wp.shape), _full(ds3_Wc.shape), _full(ds3_Wo.shape),
                  _full(s3_Wp.shape), _full(s3_Wc.shape), _full(s3_Wo.shape),
                  _full(fc_W.shape), _full((1, 384)), _full((1, 384)),
                  _full(head_W1.shape), _full((1, 1536)),
                  _full(head_W2.shape), _full((1, 250))],
        out_specs=pl.BlockSpec((_S3, 1, 250), lambda b: (b, 0, 0)),
        out_shape=jax.ShapeDtypeStruct((B, 1, 250), f32),
    )(p3, wp21, s21_Wc, s21_Wo, ds3_Wp, ds3_Wc, ds3_Wo,
      s3_Wp, s3_Wc, s3_Wo, fc_W, bn_g.reshape(1, 384), bn_b.reshape(1, 384),
      head_W1, head_b1.reshape(1, 1536), head_W2, head_b2.reshape(1, 250))

    return out.reshape(B, 250)


# speedup vs baseline: 2.1519x; 1.0407x over previous
import jax
import jax.numpy as jnp
import numpy as np
from jax.experimental import pallas as pl
from jax.experimental.pallas import tpu as pltpu

_PAR = pltpu.CompilerParams(dimension_semantics=("parallel",))


def _dot(a, b):
    return jax.lax.dot_general(a, b, (((a.ndim - 1,), (0,)), ((), ())),
                               preferred_element_type=jnp.float32)


def _dot_t(a, b):
    return jax.lax.dot_general(a, b, (((1,), (1,)), ((), ())),
                               preferred_element_type=jnp.float32)


def _topk_max(ts, t_rows, topk):
    n = ts.shape[0]
    sim = _dot_t(ts, ts)
    col = jax.lax.broadcasted_iota(jnp.int32, (n, n), 1).astype(jnp.float32)
    acc = jnp.full(t_rows.shape, -jnp.inf, dtype=jnp.float32)
    cur = sim
    for _ in range(topk):
        m = jnp.max(cur, axis=1, keepdims=True)
        w = jnp.where(cur == m, col, jnp.float32(n))
        idx = jnp.min(w, axis=1, keepdims=True)
        hit = w == idx
        acc = jnp.maximum(acc, _dot(jnp.where(hit, 1.0, 0.0), t_rows))
        cur = jnp.where(hit, -jnp.inf, cur)
    return acc


def _topk_max_stacked(t, topk, nper):
    rows = t.shape[0]
    sim = _dot_t(t, t)
    coli = jax.lax.broadcasted_iota(jnp.int32, (rows, rows), 1)
    col = coli.astype(jnp.float32)
    rowi = jax.lax.broadcasted_iota(jnp.int32, (rows, rows), 0)
    cur = jnp.where(rowi // nper == coli // nper, sim, -jnp.inf)
    acc = jnp.full(t.shape, -jnp.inf, dtype=jnp.float32)
    for _ in range(topk):
        m = jnp.max(cur, axis=1, keepdims=True)
        w = jnp.where(cur == m, col, jnp.float32(rows))
        idx = jnp.min(w, axis=1, keepdims=True)
        hit = w == idx
        acc = jnp.maximum(acc, _dot(jnp.where(hit, 1.0, 0.0), t))
        cur = jnp.where(hit, -jnp.inf, cur)
    return acc


def _tcg(patches, Wp, Wc, Wo, topk, x2, nper, stacked=False):
    t = _dot(patches, Wp)
    if x2 is not None:
        t = t + x2
    rows, d = t.shape
    if stacked:
        acc = _topk_max_stacked(t, topk, nper)
    else:
        accs = []
        for s in range(0, rows, nper):
            ts = t[s:s + nper]
            accs.append(_topk_max(ts, ts, topk))
        acc = jnp.concatenate(accs, axis=0) if len(accs) > 1 else accs[0]
    rel = acc - t
    h = _dot(t, Wc[:d]) + _dot(rel, Wc[d:])
    t_out = jax.nn.gelu(h) + t
    return _dot(t_out, Wo), t_out


_S2 = 4
_S3 = 4

_P196 = np.zeros((196, 196), dtype=np.float32)
for _a in range(7):
    for _b in range(7):
        for _pi in range(2):
            for _pj in range(2):
                _P196[(_pi * 2 + _pj) * 49 + _a * 7 + _b,
                      (2 * _a + _pi) * 14 + 2 * _b + _pj] = 1.0


def _stage12_body(p_ref, wd, d1p, d1c, d1o, s1p, s1c, s1o,
                  d2p, d2c, d2o, s2p, s2c, s2o, perm):
    s = p_ref.shape[0]
    p = jax.nn.gelu(_dot(p_ref[...].reshape(s * 196, 192), wd[...]))
    pix, x2 = _tcg(p, d1p[...], d1c[...], d1o[...], 2, None, 196)
    for i in range(5):
        pix, x2 = _tcg(pix, s1p[i], s1c[i], s1o[i], 2, x2, 196)
    pix, x2 = _tcg(pix, d2p[...], d2c[...], d2o[...], 2, None, 196)
    for i in range(2):
        pix, x2 = _tcg(pix, s2p[i], s2c[i], s2o[i], 2, x2, 196)
    pm = perm[...]
    outs = []
    for i in range(s):
        g = _dot(pm, pix[i * 196:(i + 1) * 196])
        outs.append(jnp.concatenate([g[q * 49:(q + 1) * 49] for q in range(4)],
                                    axis=1))
    return jnp.concatenate(outs, axis=0)


def _net_body(p_ref, wd, d1p, d1c, d1o, s1p, s1c, s1o,
              d2p, d2c, d2o, s2p, s2c, s2o, perm,
              s21p, s21c, s21o, d3p, d3c, d3o, s3p, s3c, s3o,
              fc, bg, bb, w1, b1, w2, b2, o_ref):
    s = p_ref.shape[0]
    mid = _stage12_body(p_ref, wd, d1p, d1c, d1o, s1p, s1c, s1o,
                        d2p, d2c, d2o, s2p, s2c, s2o, perm)
    p = mid
    pix, x2 = _tcg(p, s21p[...], s21c[...], s21o[...], 2, None, 49, True)
    pix, x2 = _tcg(pix, d3p[...], d3c[...], d3o[...], 9, None, 49, True)
    for i in range(2):
        pix, x2 = _tcg(pix, s3p[i], s3c[i], s3o[i], 2, x2, 49, True)
    f = _dot(pix, fc[...])
    f = f * bg[...] + bb[...]
    f = f * jax.nn.sigmoid(f)
    fm = jnp.mean(f.reshape(s, 49, 384), axis=1)
    h2 = jax.nn.gelu(_dot(fm, w1[...]) + b1[...])
    o_ref[...] = (_dot(h2, w2[...]) + b2[...]).reshape(s, 1, 250)


def _full(shape):
    nd = len(shape)
    return pl.BlockSpec(shape, lambda b, _nd=nd: (0,) * _nd)


def kernel(x, stem_W, ds1_Wp, ds1_Wc, ds1_Wo, s1_Wp, s1_Wc, s1_Wo,
           ds2_Wp, ds2_Wc, ds2_Wo, s20_Wp, s20_Wc, s20_Wo,
           s21_Wp, s21_Wc, s21_Wo, ds3_Wp, ds3_Wc, ds3_Wo,
           s3_Wp, s3_Wc, s3_Wo, fc_W, bn_g, bn_b,
           head_W1, head_b1, head_W2, head_b2):
    B = x.shape[0]
    f32 = jnp.float32

    sp = x.reshape(B, 3, 14, 4, 2, 14, 4, 2).transpose(0, 2, 5, 3, 6, 1, 4, 7)
    sp = sp.reshape(B, 196, 192)
    wd = jnp.kron(jnp.eye(16, dtype=f32), stem_W)
    wp1 = ds1_Wp.reshape(46, 16, 92).transpose(1, 0, 2).reshape(736, 92)

    wp21 = s21_Wp.reshape(192, 4, 192).transpose(1, 0, 2).reshape(768, 192)

    out = pl.pallas_call(
        _net_body,
        compiler_params=_PAR,
        grid=(B // _S2,),
        in_specs=[pl.BlockSpec((_S2, 196, 192), lambda b: (b, 0, 0)),
                  _full((192, 736)), _full(ds1_Wp.shape), _full(ds1_Wc.shape), _full(ds1_Wo.shape),
                  _full(s1_Wp.shape), _full(s1_Wc.shape), _full(s1_Wo.shape),
                  _full(ds2_Wp.shape), _full(ds2_Wc.shape), _full(ds2_Wo.shape),
                  _full(s20_Wp.shape), _full(s20_Wc.shape), _full(s20_Wo.shape),
                  _full((196, 196)),
                  _full(s21_Wp.shape), _full(s21_Wc.shape), _full(s21_Wo.shape),
                  _full(ds3_Wp.shape), _full(ds3_Wc.shape), _full(ds3_Wo.shape),
                  _full(s3_Wp.shape), _full(s3_Wc.shape), _full(s3_Wo.shape),
                  _full(fc_W.shape), _full((1, 384)), _full((1, 384)),
                  _full(head_W1.shape), _full((1, 1536)),
                  _full(head_W2.shape), _full((1, 250))],
        out_specs=pl.BlockSpec((_S2, 1, 250), lambda b: (b, 0, 0)),
        out_shape=jax.ShapeDtypeStruct((B, 1, 250), f32),
    )(sp, wd, wp1, ds1_Wc, ds1_Wo, s1_Wp, s1_Wc, s1_Wo,
      ds2_Wp, ds2_Wc, ds2_Wo, s20_Wp, s20_Wc, s20_Wo, jnp.asarray(_P196),
      wp21, s21_Wc, s21_Wo, ds3_Wp, ds3_Wc, ds3_Wo,
      s3_Wp, s3_Wc, s3_Wo, fc_W, bn_g.reshape(1, 384), bn_b.reshape(1, 384),
      head_W1, head_b1.reshape(1, 1536), head_W2, head_b2.reshape(1, 250))

    return out.reshape(B, 250)
